# Initial kernel scaffold; baseline (speedup 1.0000x reference)
import jax
import jax.numpy as jnp
from jax.experimental import pallas as pl


def kernel(x, edge_index, edge_weight, W0, W1):
    raise NotImplementedError("write your pallas kernel here")


# speedup vs baseline: 4.1842x; 4.1842x over previous
import jax
import jax.numpy as jnp
from jax import lax
from jax.experimental import pallas as pl
from jax.experimental.pallas import tpu as pltpu
from jax.experimental.pallas import tpu_sc as plsc

N = 10000
D_IN = 128
CHANNELS = 128
N_LABELS = 64

NC = 2
NS = 16
NW = NC * NS
CHUNK = 128
ROW_BLK = 400
GRID = N // ROW_BLK


def _sc_aggregate(C, E_pad):
  per_w = E_pad // NW
  k_chunks = per_w // CHUNK
  rows_per_s = N // NS
  nb = C // 16
  mesh = plsc.VectorSubcoreMesh(core_axis_name="c", subcore_axis_name="s",
                                num_cores=NC, num_subcores=NS)

  def body(h_hbm, src_hbm, dst_hbm, w_hbm, out_hbm,
           srcb, dstb, wb, rows, acc, sem):
    c = lax.axis_index("c")
    s = lax.axis_index("s")
    wid = s * NC + c

    def zrow(i, _):
      for j in range(nb):
        rows[i, pl.ds(16 * j, 16)] = jnp.zeros((16,), jnp.float32)
      return 0
    lax.fori_loop(0, CHUNK, zrow, 0)
    r0 = s * rows_per_s
    for k in range(rows_per_s // 125):
      pltpu.sync_copy(rows.at[pl.ds(0, 125)], acc.at[pl.ds(r0 + k * 125, 125)])
    plsc.subcore_barrier()

    base = wid * per_w

    def chunk_body(g, _):
      off = base + g * CHUNK
      pltpu.sync_copy(src_hbm.at[pl.ds(off, CHUNK)], srcb)
      pltpu.sync_copy(dst_hbm.at[pl.ds(off, CHUNK)], dstb)
      pltpu.sync_copy(w_hbm.at[pl.ds(off, CHUNK)], wb)
      pltpu.async_copy(h_hbm.at[srcb], rows, sem).wait()

      def edge_body(e, _):
        wv = plsc.load_gather(wb, [jnp.full((16,), e, jnp.int32)])
        for j in range(nb):
          sl = pl.ds(16 * j, 16)
          rows[e, sl] = rows[e, sl] * wv
        return 0
      lax.fori_loop(0, CHUNK, edge_body, 0)

      pltpu.sync_copy(rows, acc.at[dstb], add=True)
      return 0

    lax.fori_loop(0, k_chunks, chunk_body, 0)

    plsc.subcore_barrier()
    pltpu.sync_copy(acc.at[pl.ds(r0, rows_per_s)],
                    out_hbm.at[c, pl.ds(r0, rows_per_s)])

  return pl.kernel(
      body,
      out_type=jax.ShapeDtypeStruct((NC, N, C), jnp.float32),
      mesh=mesh,
      compiler_params=pltpu.CompilerParams(use_tc_tiling_on_sc=False,
                                           needs_layout_passes=False),
      scratch_types=[
          pltpu.VMEM((CHUNK,), jnp.int32),
          pltpu.VMEM((CHUNK,), jnp.int32),
          pltpu.VMEM((CHUNK,), jnp.float32),
          pltpu.VMEM((CHUNK, C), jnp.float32),
          pltpu.VMEM_SHARED((N, C), jnp.float32),
          pltpu.SemaphoreType.DMA,
      ],
  )


def _tc_matmul(x, w):
  K, C = w.shape

  def body(x_ref, w_ref, o_ref):
    o_ref[...] = jnp.dot(x_ref[...], w_ref[...],
                         preferred_element_type=jnp.float32)

  return pl.pallas_call(
      body,
      grid=(GRID,),
      in_specs=[pl.BlockSpec((ROW_BLK, K), lambda i: (i, 0)),
                pl.BlockSpec((K, C), lambda i: (0, 0))],
      out_specs=pl.BlockSpec((ROW_BLK, C), lambda i: (i, 0)),
      out_shape=jax.ShapeDtypeStruct((N, C), jnp.float32),
  )(x, w)


def _tc_add_relu_matmul(p, w):
  K, C = w.shape

  def body(p_ref, w_ref, o_ref):
    h = jnp.maximum(p_ref[0] + p_ref[1], 0.0)
    o_ref[...] = jnp.dot(h, w_ref[...], preferred_element_type=jnp.float32)

  return pl.pallas_call(
      body,
      grid=(GRID,),
      in_specs=[pl.BlockSpec((NC, ROW_BLK, K), lambda i: (0, i, 0)),
                pl.BlockSpec((K, C), lambda i: (0, 0))],
      out_specs=pl.BlockSpec((ROW_BLK, C), lambda i: (i, 0)),
      out_shape=jax.ShapeDtypeStruct((N, C), jnp.float32),
  )(p, w)


def _tc_add_softmax(q):
  C = q.shape[-1]

  def body(q_ref, o_ref):
    z = q_ref[0] + q_ref[1]
    m = jnp.max(z, axis=-1, keepdims=True)
    e = jnp.exp(z - m)
    o_ref[...] = e / jnp.sum(e, axis=-1, keepdims=True)

  return pl.pallas_call(
      body,
      grid=(GRID,),
      in_specs=[pl.BlockSpec((NC, ROW_BLK, C), lambda i: (0, i, 0))],
      out_specs=pl.BlockSpec((ROW_BLK, C), lambda i: (i, 0)),
      out_shape=jax.ShapeDtypeStruct((N, C), jnp.float32),
  )(q)


def kernel(x, edge_index, edge_weight, W0, W1):
  E = edge_weight.shape[0]
  per_w = -(-E // (NW * CHUNK)) * CHUNK
  E_pad = NW * per_w
  pad = E_pad - E
  src = edge_index[0]
  dst = edge_index[1]
  w = edge_weight
  if pad:
    fill = (jnp.arange(pad, dtype=jnp.int32) * 37) % N
    src = jnp.concatenate([src, fill])
    dst = jnp.concatenate([dst, fill])
    w = jnp.concatenate([w, jnp.zeros((pad,), jnp.float32)])

  h0 = _tc_matmul(x, W0)
  p = _sc_aggregate(CHANNELS, E_pad)(h0, src, dst, w)
  h1 = _tc_add_relu_matmul(p, W1)
  q = _sc_aggregate(N_LABELS, E_pad)(h1, src, dst, w)
  return _tc_add_softmax(q)


# speedup vs baseline: 9.7789x; 2.3371x over previous
import jax
import jax.numpy as jnp
from jax import lax
from jax.experimental import pallas as pl
from jax.experimental.pallas import tpu as pltpu
from jax.experimental.pallas import tpu_sc as plsc

N = 10000
D_IN = 128
CHANNELS = 128
N_LABELS = 64

NC = 2
NS = 16
NW = NC * NS
ROW_BLK = 400
GRID = N // ROW_BLK


NSLOT = 4


def _sc_aggregate(C, E_pad, CHUNK):
  per_w = E_pad // NW
  K = per_w // CHUNK
  assert K % 4 == 0 and K >= 8
  rows_per_s = N // NS
  nb = C // 16
  mesh = plsc.VectorSubcoreMesh(core_axis_name="c", subcore_axis_name="s",
                                num_cores=NC, num_subcores=NS)

  def body(h_hbm, sw_hbm, dst_hbm, out_hbm, *scr):
    swb = scr[0:NSLOT]
    dstb = scr[NSLOT:2 * NSLOT]
    rows = scr[2 * NSLOT:3 * NSLOT]
    acc = scr[3 * NSLOT]
    sw_sem = scr[3 * NSLOT + 1:3 * NSLOT + 1 + NSLOT]
    d_sem = scr[3 * NSLOT + 1 + NSLOT:3 * NSLOT + 1 + 2 * NSLOT]
    g_sem = scr[3 * NSLOT + 1 + 2 * NSLOT:3 * NSLOT + 1 + 3 * NSLOT]
    s_sem = scr[3 * NSLOT + 1 + 3 * NSLOT:3 * NSLOT + 1 + 4 * NSLOT]

    c = lax.axis_index("c")
    s = lax.axis_index("s")
    wid = s * NC + c
    base_chunk = wid * K

    def issue_sw(g, u):
      pltpu.async_copy(sw_hbm.at[base_chunk + g], swb[u], sw_sem[u])

    def wait_sw(g, u):
      pltpu.make_async_copy(sw_hbm.at[base_chunk + g], swb[u], sw_sem[u]).wait()

    def issue_dst(g, u):
      pltpu.async_copy(dst_hbm.at[pl.ds((base_chunk + g) * CHUNK, CHUNK)],
                       dstb[u], d_sem[u])

    def wait_dst(g, u):
      pltpu.make_async_copy(dst_hbm.at[pl.ds((base_chunk + g) * CHUNK, CHUNK)],
                            dstb[u], d_sem[u]).wait()

    def issue_gather(u):
      pltpu.async_copy(h_hbm.at[swb[u].at[0]], rows[u], g_sem[u])

    def wait_gather(u):
      pltpu.make_async_copy(h_hbm.at[swb[u].at[0]], rows[u], g_sem[u]).wait()

    def issue_scatter(u):
      pltpu.async_copy(rows[u], acc.at[dstb[u]], s_sem[u], add=True)

    def wait_scatter(u):
      pltpu.make_async_copy(rows[u], acc.at[dstb[u]], s_sem[u]).wait()

    def scale(u):
      rbuf = rows[u]
      wrow = swb[u]

      def edge2(i, _):
        e0 = i * 2
        wv = plsc.bitcast(wrow[1, pl.ds(e0 & ~15, 16)], jnp.float32)
        for d in range(2):
          e = e0 + d
          idx = jnp.full((16, 1), e & 15, jnp.int32)
          ws = lax.gather(
              wv, idx,
              dimension_numbers=lax.GatherDimensionNumbers(
                  offset_dims=(), collapsed_slice_dims=(0,),
                  start_index_map=(0,)),
              slice_sizes=(1,),
              mode=lax.GatherScatterMode.PROMISE_IN_BOUNDS)
          for j in range(nb):
            sl = pl.ds(16 * j, 16)
            rbuf[e, sl] = rbuf[e, sl] * ws
        return 0

      lax.fori_loop(0, CHUNK // 2, edge2, 0)

    def pre(g, u):
      wait_sw(g, u)
      issue_dst(g, u)
      issue_gather(u)

    def post(g, u, refill):
      wait_gather(u)
      scale(u)
      wait_dst(g, u)
      issue_scatter(u)
      if isinstance(refill, int):
        if refill < K:
          issue_sw(refill, u)
      else:
        @pl.when(refill < K)
        def _():
          issue_sw(refill, u)

    for u in range(NSLOT):
      issue_sw(u, u)

    def zrow(i, _):
      for j in range(nb):
        rows[0][i, pl.ds(16 * j, 16)] = jnp.zeros((16,), jnp.float32)
      return 0
    lax.fori_loop(0, CHUNK, zrow, 0)
    r0 = s * rows_per_s
    off = 0
    while off < rows_per_s:
      n = min(CHUNK, rows_per_s - off)
      pltpu.sync_copy(rows[0].at[pl.ds(0, n)],
                      acc.at[pl.ds(r0 + off, n)])
      off += n
    plsc.subcore_barrier()

    pre(0, 0)
    pre(1, 1)
    post(0, 0, 4)
    pre(2, 2)
    post(1, 1, 5)
    pre(3, 3)
    post(2, 2, 6)

    def steady(t, _):
      g0 = 4 * t
      for u in range(NSLOT):
        g = g0 + u
        wait_scatter(u)
        pre(g, u)
        pu = (u + 3) % 4
        post(g - 1, pu, g + 3)
      return 0

    lax.fori_loop(1, K // 4, steady, 0)

    post(K - 1, 3, K + 3)
    for u in range(NSLOT):
      wait_scatter(u)

    plsc.subcore_barrier()
    pltpu.sync_copy(acc.at[pl.ds(r0, rows_per_s)],
                    out_hbm.at[c, pl.ds(r0, rows_per_s)])

  return pl.kernel(
      body,
      out_type=jax.ShapeDtypeStruct((NC, N, C), jnp.float32),
      mesh=mesh,
      compiler_params=pltpu.CompilerParams(use_tc_tiling_on_sc=False,
                                           needs_layout_passes=False),
      scratch_types=(
          [pltpu.VMEM((2, CHUNK), jnp.int32) for _ in range(NSLOT)]
          + [pltpu.VMEM((CHUNK,), jnp.int32) for _ in range(NSLOT)]
          + [pltpu.VMEM((CHUNK, C), jnp.float32) for _ in range(NSLOT)]
          + [pltpu.VMEM_SHARED((N, C), jnp.float32)]
          + [pltpu.SemaphoreType.DMA for _ in range(4 * NSLOT)]
      ),
  )


def _tc_matmul(x, w):
  K, C = w.shape

  def body(x_ref, w_ref, o_ref):
    o_ref[...] = jnp.dot(x_ref[...], w_ref[...],
                         preferred_element_type=jnp.float32)

  return pl.pallas_call(
      body,
      grid=(GRID,),
      in_specs=[pl.BlockSpec((ROW_BLK, K), lambda i: (i, 0)),
                pl.BlockSpec((K, C), lambda i: (0, 0))],
      out_specs=pl.BlockSpec((ROW_BLK, C), lambda i: (i, 0)),
      out_shape=jax.ShapeDtypeStruct((N, C), jnp.float32),
  )(x, w)


def _tc_add_relu_matmul(p, w):
  K, C = w.shape

  def body(p_ref, w_ref, o_ref):
    h = jnp.maximum(p_ref[0] + p_ref[1], 0.0)
    o_ref[...] = jnp.dot(h, w_ref[...], preferred_element_type=jnp.float32)

  return pl.pallas_call(
      body,
      grid=(GRID,),
      in_specs=[pl.BlockSpec((NC, ROW_BLK, K), lambda i: (0, i, 0)),
                pl.BlockSpec((K, C), lambda i: (0, 0))],
      out_specs=pl.BlockSpec((ROW_BLK, C), lambda i: (i, 0)),
      out_shape=jax.ShapeDtypeStruct((N, C), jnp.float32),
  )(p, w)


def _tc_add_softmax(q):
  C = q.shape[-1]

  def body(q_ref, o_ref):
    z = q_ref[0] + q_ref[1]
    m = jnp.max(z, axis=-1, keepdims=True)
    e = jnp.exp(z - m)
    o_ref[...] = e / jnp.sum(e, axis=-1, keepdims=True)

  return pl.pallas_call(
      body,
      grid=(GRID,),
      in_specs=[pl.BlockSpec((NC, ROW_BLK, C), lambda i: (0, i, 0))],
      out_specs=pl.BlockSpec((ROW_BLK, C), lambda i: (i, 0)),
      out_shape=jax.ShapeDtypeStruct((N, C), jnp.float32),
  )(q)


def kernel(x, edge_index, edge_weight, W0, W1):
  E = edge_weight.shape[0]
  per_w = -(-E // (NW * 512)) * 512
  E_pad = NW * per_w
  pad = E_pad - E
  src = edge_index[0]
  dst = edge_index[1]
  w = edge_weight
  if pad:
    fill = (jnp.arange(pad, dtype=jnp.int32) * 37) % N
    src = jnp.concatenate([src, fill])
    dst = jnp.concatenate([dst, fill])
    w = jnp.concatenate([w, jnp.zeros((pad,), jnp.float32)])

  wbits = jax.lax.bitcast_convert_type(w, jnp.int32)

  def swpack(chunk):
    return jnp.stack([src.reshape(-1, chunk), wbits.reshape(-1, chunk)],
                     axis=1)

  h0 = _tc_matmul(x, W0)
  p = _sc_aggregate(CHANNELS, E_pad, 64)(h0, swpack(64), dst)
  h1 = _tc_add_relu_matmul(p, W1)
  q = _sc_aggregate(N_LABELS, E_pad, 128)(h1, swpack(128), dst)
  return _tc_add_softmax(q)
